# baseline (speedup 1.0000x reference)
import functools

import jax
import jax.numpy as jnp
from jax import lax
from jax.experimental import pallas as pl
from jax.experimental.pallas import tpu as pltpu
from jax.experimental.pallas import tpu_sc as plsc

N = 10000
E = 320000
DPAD = 16
NT = 16
NN = 10240
NPT = NN // NT

E2 = E + N
NCH2 = 162
EPT2 = NCH2 * 128
EPAD2 = NT * EPT2

NCH = 158
EPT = NCH * 128
EPAD = NT * EPT

_f32 = jnp.float32


def _prep_from_x(x_ref, w_ref, as_ref, ad_ref, hp_ref, hs_ref, hd_ref, c_ref, *, d_out):
    h = jnp.dot(x_ref[...], w_ref[...], preferred_element_type=_f32)
    col = lax.broadcasted_iota(jnp.int32, (N, DPAD), 1)
    hp_ref[:N, :] = h + jnp.where(col == d_out, 1.0, 0.0).astype(_f32)
    hp_ref[N:, :] = jnp.zeros((NN - N, DPAD), _f32)
    hs = jnp.dot(h, as_ref[...], preferred_element_type=_f32)
    hd = jnp.dot(h, ad_ref[...], preferred_element_type=_f32)
    hs_ref[:N, :] = hs
    hs_ref[N:, :] = jnp.zeros((NN - N, 1), _f32)
    hd_ref[:N, :] = hd
    hd_ref[N:, :] = jnp.zeros((NN - N, 1), _f32)
    stab = jnp.max(hs) + hd
    c_ref[:N, :] = jnp.maximum(stab, 0.2 * stab)
    c_ref[N:, :] = jnp.zeros((NN - N, 1), _f32)


def _prep_from_acc(acc_ref, b_ref, w_ref, as_ref, ad_ref, hp_ref, hs_ref, hd_ref,
                   c_ref, *, d_prev, d_out):
    num = acc_ref[:N, :]
    den = num[:, d_prev:d_prev + 1] + 1e-16
    hprev = jnp.maximum(num / den + b_ref[...], 0.0)
    h = jnp.dot(hprev, w_ref[...], preferred_element_type=_f32)
    col = lax.broadcasted_iota(jnp.int32, (N, DPAD), 1)
    hp_ref[:N, :] = h + jnp.where(col == d_out, 1.0, 0.0).astype(_f32)
    hp_ref[N:, :] = jnp.zeros((NN - N, DPAD), _f32)
    hs = jnp.dot(h, as_ref[...], preferred_element_type=_f32)
    hd = jnp.dot(h, ad_ref[...], preferred_element_type=_f32)
    hs_ref[:N, :] = hs
    hs_ref[N:, :] = jnp.zeros((NN - N, 1), _f32)
    hd_ref[:N, :] = hd
    hd_ref[N:, :] = jnp.zeros((NN - N, 1), _f32)
    stab = jnp.max(hs) + hd
    c_ref[:N, :] = jnp.maximum(stab, 0.2 * stab)
    c_ref[N:, :] = jnp.zeros((NN - N, 1), _f32)


def _prep_final(acc_ref, b_ref, wa_ref, wb_ref, be_ref, p_ref, q_ref, *, d_prev):
    num = acc_ref[:N, :]
    den = num[:, d_prev:d_prev + 1] + 1e-16
    h = jnp.maximum(num / den + b_ref[...], 0.0)
    p_ref[:N, :] = jnp.dot(h, wa_ref[...], preferred_element_type=_f32) + be_ref[...]
    p_ref[N:, :] = jnp.zeros((NN - N, DPAD), _f32)
    q_ref[:N, :] = jnp.dot(h, wb_ref[...], preferred_element_type=_f32)
    q_ref[N:, :] = jnp.zeros((NN - N, DPAD), _f32)


def _tc_prep_x(x, wp, asp, adp, d_out):
    return pl.pallas_call(
        functools.partial(_prep_from_x, d_out=d_out),
        out_shape=[
            jax.ShapeDtypeStruct((NN, DPAD), _f32),
            jax.ShapeDtypeStruct((NN, 1), _f32),
            jax.ShapeDtypeStruct((NN, 1), _f32),
            jax.ShapeDtypeStruct((NN, 1), _f32),
        ],
    )(x, wp, asp, adp)


def _tc_prep_acc(acc, bp, wp, asp, adp, d_prev, d_out):
    return pl.pallas_call(
        functools.partial(_prep_from_acc, d_prev=d_prev, d_out=d_out),
        out_shape=[
            jax.ShapeDtypeStruct((NN, DPAD), _f32),
            jax.ShapeDtypeStruct((NN, 1), _f32),
            jax.ShapeDtypeStruct((NN, 1), _f32),
            jax.ShapeDtypeStruct((NN, 1), _f32),
        ],
    )(acc, bp, wp, asp, adp)


def _tc_prep_final(acc, bp, wap, wbp, bep, d_prev):
    return pl.pallas_call(
        functools.partial(_prep_final, d_prev=d_prev),
        out_shape=[
            jax.ShapeDtypeStruct((NN, DPAD), _f32),
            jax.ShapeDtypeStruct((NN, DPAD), _f32),
        ],
    )(acc, bp, wap, wbp, bep)


def _make_gat_edge_kernel():
    mesh = plsc.VectorSubcoreMesh(
        core_axis_name="c", subcore_axis_name="s", num_cores=1)

    @functools.partial(
        pl.kernel, mesh=mesh,
        compiler_params=pltpu.CompilerParams(
            needs_layout_passes=False, use_tc_tiling_on_sc=False),
        out_type=jax.ShapeDtypeStruct((NN, DPAD), _f32),
        scratch_types=[
            pltpu.VMEM((NN,), _f32),
            pltpu.VMEM((NN,), _f32),
            pltpu.VMEM((NN,), _f32),
            pltpu.VMEM((NCH2, 128), jnp.int32),
            pltpu.VMEM((NCH2, 128), jnp.int32),
            pltpu.VMEM((128, DPAD), _f32),
            pltpu.VMEM((NPT, DPAD), _f32),
            pltpu.VMEM_SHARED((NN, DPAD), _f32),
            pltpu.VMEM_SHARED((NN, DPAD), _f32),
            pltpu.SemaphoreType.DMA,
        ],
    )
    def k(hp_hbm, hs_hbm, hd_hbm, c_hbm, s3_hbm, d3_hbm, out_hbm,
          hs_v, hd_v, c_v, s3v, d3v, rows_v, z_v, hsp, accsp, sem):
        sub = lax.axis_index("s")
        i16 = lax.iota(jnp.int32, 16)
        zero16 = jnp.zeros((16,), _f32)

        pltpu.sync_copy(hs_hbm, hs_v)
        pltpu.sync_copy(hd_hbm, hd_v)
        pltpu.sync_copy(c_hbm, c_v)
        pltpu.sync_copy(s3_hbm.at[sub], s3v)
        pltpu.sync_copy(d3_hbm.at[sub], d3v)
        nslice = pl.ds(sub * NPT, NPT)
        pltpu.sync_copy(hp_hbm.at[nslice], hsp.at[nslice])
        for r in range(NPT):
            z_v[r, :] = zero16
        pltpu.sync_copy(z_v, accsp.at[nslice])
        plsc.subcore_barrier()

        ebase = sub * EPT2

        def chunk_body(j, carry):
            pltpu.async_copy(hsp.at[s3v.at[j]], rows_v, sem).wait()
            for g in range(8):
                s16 = s3v[j, pl.ds(g * 16, 16)]
                d16 = d3v[j, pl.ds(g * 16, 16)]
                hs_g = plsc.load_gather(hs_v, [s16])
                hd_g = plsc.load_gather(hd_v, [d16])
                c_g = plsc.load_gather(c_v, [d16])
                z = hs_g + hd_g
                lg = jnp.maximum(z, 0.2 * z)
                ex = jnp.exp(lg - c_g)
                eid = ebase + j * 128 + g * 16 + i16
                ex = jnp.where(eid < E2, ex, 0.0)
                for kk in range(16):
                    r = g * 16 + kk
                    exk = jnp.broadcast_to(ex[kk], (16,))
                    rows_v[r, :] = rows_v[r, :] * exk
            pltpu.sync_copy(rows_v, accsp.at[d3v.at[j]], add=True)
            return carry

        lax.fori_loop(0, NCH2, chunk_body, 0)
        plsc.subcore_barrier()
        pltpu.sync_copy(accsp.at[nslice], out_hbm.at[nslice])

    return k


def _make_edge_head_kernel():
    mesh = plsc.VectorSubcoreMesh(
        core_axis_name="c", subcore_axis_name="s", num_cores=1)

    @functools.partial(
        pl.kernel, mesh=mesh,
        compiler_params=pltpu.CompilerParams(
            needs_layout_passes=False, use_tc_tiling_on_sc=False),
        out_type=jax.ShapeDtypeStruct((EPAD * 4,), _f32),
        scratch_types=[
            pltpu.VMEM((NCH, 128), jnp.int32),
            pltpu.VMEM((NCH, 128), jnp.int32),
            pltpu.VMEM((128, DPAD), _f32),
            pltpu.VMEM((128, DPAD), _f32),
            pltpu.VMEM((2048,), _f32),
            pltpu.VMEM((64,), _f32),
            pltpu.VMEM((16,), _f32),
            pltpu.VMEM((512,), _f32),
            pltpu.VMEM_SHARED((NN, DPAD), _f32),
            pltpu.VMEM_SHARED((NN, DPAD), _f32),
            pltpu.SemaphoreType.DMA,
        ],
    )
    def k(p_hbm, q_hbm, s3_hbm, d3_hbm, w9_hbm, b9_hbm, out_hbm,
          s3v, d3v, pr_v, qr_v, ut_v, w9_v, b9_v, ob_v, psp, qsp, sem):
        sub = lax.axis_index("s")
        i16 = lax.iota(jnp.int32, 16)

        pltpu.sync_copy(s3_hbm.at[sub], s3v)
        pltpu.sync_copy(d3_hbm.at[sub], d3v)
        pltpu.sync_copy(w9_hbm, w9_v)
        pltpu.sync_copy(b9_hbm, b9_v)
        nslice = pl.ds(sub * NPT, NPT)
        pltpu.sync_copy(p_hbm.at[nslice], psp.at[nslice])
        pltpu.sync_copy(q_hbm.at[nslice], qsp.at[nslice])
        plsc.subcore_barrier()

        w9cols = [w9_v[pl.ds(c * 16, 16)] for c in range(4)]
        b9all = b9_v[...]
        w9b = [[jnp.broadcast_to(w9cols[c][f], (16,)) for c in range(4)]
               for f in range(10)]
        b9b = [jnp.broadcast_to(b9all[c], (16,)) for c in range(4)]
        i16x128 = i16 * 128
        obase = sub * EPT * 4

        def chunk_body(j, carry):
            pltpu.async_copy(psp.at[d3v.at[j]], pr_v, sem).wait()
            pltpu.async_copy(qsp.at[s3v.at[j]], qr_v, sem).wait()
            for r in range(128):
                u = jnp.maximum(pr_v[r, :] + qr_v[r, :], 0.0)
                plsc.store_scatter(ut_v, [i16x128 + r], u)
            for g in range(8):
                o = [b9b[c] for c in range(4)]
                for f in range(10):
                    uf = ut_v[pl.ds(f * 128 + g * 16, 16)]
                    o = [o[c] + uf * w9b[f][c] for c in range(4)]
                o = [jnp.maximum(oc, 0.0) for oc in o]
                m = jnp.maximum(jnp.maximum(o[0], o[1]), jnp.maximum(o[2], o[3]))
                ev = [jnp.exp(oc - m) for oc in o]
                r = 1.0 / (ev[0] + ev[1] + ev[2] + ev[3])
                le4 = g * 64 + i16 * 4
                for c in range(4):
                    plsc.store_scatter(ob_v, [le4 + c], ev[c] * r)
            pltpu.sync_copy(ob_v, out_hbm.at[pl.ds(obase + j * 512, 512)])
            return carry

        lax.fori_loop(0, NCH, chunk_body, 0)

    return k


_gat = _make_gat_edge_kernel()
_head = _make_edge_head_kernel()


def _padw(w, r, c):
    return jnp.zeros((r, c), _f32).at[:w.shape[0], :w.shape[1]].set(w)


def _padv(v, r):
    return jnp.zeros((r,), _f32).at[:v.shape[0]].set(v)


def kernel(x, e, W1, a1s, a1d, b1, W2, a2s, a2d, b2, W3, a3s, a3d, b3, We, be,
           W9, b9, edge_index):
    si = jnp.arange(N, dtype=jnp.int32)
    s2_full = jnp.zeros((EPAD2,), jnp.int32).at[:E].set(edge_index[0]).at[E:E2].set(si)
    d2_full = jnp.zeros((EPAD2,), jnp.int32).at[:E].set(edge_index[1]).at[E:E2].set(si)
    sa3 = s2_full.reshape(NT, NCH2, 128)
    da3 = d2_full.reshape(NT, NCH2, 128)

    s_flat = jnp.zeros((EPAD,), jnp.int32).at[:E].set(edge_index[0])
    d_flat = jnp.zeros((EPAD,), jnp.int32).at[:E].set(edge_index[1])
    s3 = s_flat.reshape(NT, NCH, 128)
    d3 = d_flat.reshape(NT, NCH, 128)

    hp, hs, hd, c = _tc_prep_x(
        x, _padw(W1, 128, DPAD), _padw(a1s[:, None], DPAD, 1),
        _padw(a1d[:, None], DPAD, 1), 5)
    acc = _gat(hp, hs.reshape(NN), hd.reshape(NN), c.reshape(NN), sa3, da3)

    hp, hs, hd, c = _tc_prep_acc(
        acc, _padv(b1, DPAD)[None, :], _padw(W2, DPAD, DPAD),
        _padw(a2s[:, None], DPAD, 1), _padw(a2d[:, None], DPAD, 1), 5, 10)
    acc = _gat(hp, hs.reshape(NN), hd.reshape(NN), c.reshape(NN), sa3, da3)

    hp, hs, hd, c = _tc_prep_acc(
        acc, _padv(b2, DPAD)[None, :], _padw(W3, DPAD, DPAD),
        _padw(a3s[:, None], DPAD, 1), _padw(a3d[:, None], DPAD, 1), 10, 10)
    acc = _gat(hp, hs.reshape(NN), hd.reshape(NN), c.reshape(NN), sa3, da3)

    p, q = _tc_prep_final(
        acc, _padv(b3, DPAD)[None, :], _padw(We[:10] - We[10:], DPAD, DPAD),
        _padw(We[10:], DPAD, DPAD), _padv(be, DPAD)[None, :], 10)
    w9cols = jnp.zeros((64,), _f32).at[0:10].set(W9[:, 0]).at[16:26].set(
        W9[:, 1]).at[32:42].set(W9[:, 2]).at[48:58].set(W9[:, 3])
    out = _head(p, q, s3, d3, w9cols, _padv(b9, 16))
    return out.reshape(EPAD, 4)[:E]


# speedup vs baseline: 1.3204x; 1.3204x over previous
import functools

import jax
import jax.numpy as jnp
from jax import lax
from jax.experimental import pallas as pl
from jax.experimental.pallas import tpu as pltpu
from jax.experimental.pallas import tpu_sc as plsc

N = 10000
E = 320000
DPAD = 16
NW = 32
NN = 10240
NPT = NN // 16

E2 = E + N
NCH2 = 82
EPT2 = NCH2 * 128
EPAD2 = NW * EPT2

NCH = 80
EPT = NCH * 128
EPAD = NW * EPT

_f32 = jnp.float32


def _emit_node_tables(h, as_ref, ad_ref, hp_ref, hs_ref, hd_ref, c_ref, d_out):
    col = lax.broadcasted_iota(jnp.int32, (N, DPAD), 1)
    hp_ref[:N, :] = h + jnp.where(col == d_out, 1.0, 0.0).astype(_f32)
    hp_ref[N:, :] = jnp.zeros((NN - N, DPAD), _f32)
    hs = jnp.dot(h, as_ref[...], preferred_element_type=_f32)
    hd = jnp.dot(h, ad_ref[...], preferred_element_type=_f32)
    hs_ref[:N, :] = hs
    hs_ref[N:, :] = jnp.zeros((NN - N, 1), _f32)
    hd_ref[:N, :] = hd
    hd_ref[N:, :] = jnp.zeros((NN - N, 1), _f32)
    stab = jnp.max(hs) + hd
    c_ref[:N, :] = jnp.maximum(stab, 0.2 * stab)
    c_ref[N:, :] = jnp.zeros((NN - N, 1), _f32)


def _prep_from_x(x_ref, w_ref, as_ref, ad_ref, hp_ref, hs_ref, hd_ref, c_ref, *, d_out):
    h = jnp.dot(x_ref[...], w_ref[...], preferred_element_type=_f32)
    _emit_node_tables(h, as_ref, ad_ref, hp_ref, hs_ref, hd_ref, c_ref, d_out)


def _prep_from_acc(acc_ref, b_ref, w_ref, as_ref, ad_ref, hp_ref, hs_ref, hd_ref,
                   c_ref, *, d_prev, d_out):
    num = acc_ref[:N, :] + acc_ref[NN:NN + N, :]
    den = num[:, d_prev:d_prev + 1] + 1e-16
    hprev = jnp.maximum(num / den + b_ref[...], 0.0)
    h = jnp.dot(hprev, w_ref[...], preferred_element_type=_f32)
    _emit_node_tables(h, as_ref, ad_ref, hp_ref, hs_ref, hd_ref, c_ref, d_out)


def _prep_final(acc_ref, b_ref, wa_ref, wb_ref, be_ref, p_ref, q_ref, *, d_prev):
    num = acc_ref[:N, :] + acc_ref[NN:NN + N, :]
    den = num[:, d_prev:d_prev + 1] + 1e-16
    h = jnp.maximum(num / den + b_ref[...], 0.0)
    p_ref[:N, :] = jnp.dot(h, wa_ref[...], preferred_element_type=_f32) + be_ref[...]
    p_ref[N:, :] = jnp.zeros((NN - N, DPAD), _f32)
    q_ref[:N, :] = jnp.dot(h, wb_ref[...], preferred_element_type=_f32)
    q_ref[N:, :] = jnp.zeros((NN - N, DPAD), _f32)


_TABLE_OUT = [
    jax.ShapeDtypeStruct((NN, DPAD), _f32),
    jax.ShapeDtypeStruct((NN, 1), _f32),
    jax.ShapeDtypeStruct((NN, 1), _f32),
    jax.ShapeDtypeStruct((NN, 1), _f32),
]


def _tc_prep_x(x, wp, asp, adp, d_out):
    return pl.pallas_call(
        functools.partial(_prep_from_x, d_out=d_out),
        out_shape=_TABLE_OUT,
    )(x, wp, asp, adp)


def _tc_prep_acc(acc, bp, wp, asp, adp, d_prev, d_out):
    return pl.pallas_call(
        functools.partial(_prep_from_acc, d_prev=d_prev, d_out=d_out),
        out_shape=_TABLE_OUT,
    )(acc, bp, wp, asp, adp)


def _tc_prep_final(acc, bp, wap, wbp, bep, d_prev):
    return pl.pallas_call(
        functools.partial(_prep_final, d_prev=d_prev),
        out_shape=[
            jax.ShapeDtypeStruct((NN, DPAD), _f32),
            jax.ShapeDtypeStruct((NN, DPAD), _f32),
        ],
    )(acc, bp, wap, wbp, bep)


def _make_gat_edge_kernel():
    mesh = plsc.VectorSubcoreMesh(core_axis_name="c", subcore_axis_name="s")

    @functools.partial(
        pl.kernel, mesh=mesh,
        compiler_params=pltpu.CompilerParams(
            needs_layout_passes=False, use_tc_tiling_on_sc=False),
        out_type=jax.ShapeDtypeStruct((2 * NN, DPAD), _f32),
        scratch_types=[
            pltpu.VMEM((NN,), _f32),
            pltpu.VMEM((NN,), _f32),
            pltpu.VMEM((NN,), _f32),
            pltpu.VMEM((NCH2, 128), jnp.int32),
            pltpu.VMEM((NCH2, 128), jnp.int32),
            pltpu.VMEM((128, DPAD), _f32),
            pltpu.VMEM((128, DPAD), _f32),
            pltpu.VMEM((NPT, DPAD), _f32),
            pltpu.VMEM_SHARED((NN, DPAD), _f32),
            pltpu.VMEM_SHARED((NN, DPAD), _f32),
            pltpu.SemaphoreType.DMA,
            pltpu.SemaphoreType.DMA,
        ],
    )
    def k(hp_hbm, hs_hbm, hd_hbm, c_hbm, s3_hbm, d3_hbm, out_hbm,
          hs_v, hd_v, c_v, s3v, d3v, rows_a, rows_b, z_v, hsp, accsp,
          sem_a, sem_b):
        core = lax.axis_index("c")
        sub = lax.axis_index("s")
        wid = sub * 2 + core
        i16 = lax.iota(jnp.int32, 16)
        zero16 = jnp.zeros((16,), _f32)

        pltpu.sync_copy(hs_hbm, hs_v)
        pltpu.sync_copy(hd_hbm, hd_v)
        pltpu.sync_copy(c_hbm, c_v)
        pltpu.sync_copy(s3_hbm.at[wid], s3v)
        pltpu.sync_copy(d3_hbm.at[wid], d3v)
        nslice = pl.ds(sub * NPT, NPT)
        pltpu.sync_copy(hp_hbm.at[nslice], hsp.at[nslice])
        for r in range(NPT):
            z_v[r, :] = zero16
        pltpu.sync_copy(z_v, accsp.at[nslice])
        plsc.subcore_barrier()

        ebase = wid * EPT2
        bufs = (rows_a, rows_b)
        sems = (sem_a, sem_b)

        def do_chunk(j, rows_v, sem):
            pltpu.make_async_copy(hsp.at[s3v.at[j]], rows_v, sem).wait()
            for g in range(8):
                s16 = s3v[j, pl.ds(g * 16, 16)]
                d16 = d3v[j, pl.ds(g * 16, 16)]
                hs_g = plsc.load_gather(hs_v, [s16])
                hd_g = plsc.load_gather(hd_v, [d16])
                c_g = plsc.load_gather(c_v, [d16])
                z = hs_g + hd_g
                lg = jnp.maximum(z, 0.2 * z)
                ex = jnp.exp(lg - c_g)
                eid = ebase + j * 128 + g * 16 + i16
                ex = jnp.where(eid < E2, ex, 0.0)
                row16 = g * 16 + i16
                for f in range(11):
                    colf = jnp.full((16,), f, jnp.int32)
                    v = plsc.load_gather(rows_v, [row16, colf])
                    plsc.store_scatter(rows_v, [row16, colf], v * ex)
            pltpu.sync_copy(rows_v, accsp.at[d3v.at[j]], add=True)

        pltpu.async_copy(hsp.at[s3v.at[0]], rows_a, sem_a)

        def pair_body(jj, carry):
            pltpu.async_copy(hsp.at[s3v.at[jj + 1]], rows_b, sem_b)
            do_chunk(jj, rows_a, sem_a)

            @pl.when(jj + 2 < NCH2)
            def _():
                pltpu.async_copy(hsp.at[s3v.at[jj + 2]], rows_a, sem_a)

            do_chunk(jj + 1, rows_b, sem_b)
            return carry

        lax.fori_loop(0, NCH2 // 2, lambda i, c: pair_body(i * 2, c), 0)
        plsc.subcore_barrier()
        pltpu.sync_copy(accsp.at[nslice],
                        out_hbm.at[pl.ds(core * NN + sub * NPT, NPT)])

    return k


def _make_edge_head_kernel():
    mesh = plsc.VectorSubcoreMesh(core_axis_name="c", subcore_axis_name="s")

    @functools.partial(
        pl.kernel, mesh=mesh,
        compiler_params=pltpu.CompilerParams(
            needs_layout_passes=False, use_tc_tiling_on_sc=False),
        out_type=jax.ShapeDtypeStruct((EPAD * 4,), _f32),
        scratch_types=[
            pltpu.VMEM((NCH, 128), jnp.int32),
            pltpu.VMEM((NCH, 128), jnp.int32),
            pltpu.VMEM((128, DPAD), _f32),
            pltpu.VMEM((128, DPAD), _f32),
            pltpu.VMEM((128, DPAD), _f32),
            pltpu.VMEM((128, DPAD), _f32),
            pltpu.VMEM((64,), _f32),
            pltpu.VMEM((16,), _f32),
            pltpu.VMEM((512,), _f32),
            pltpu.VMEM((512,), _f32),
            pltpu.VMEM_SHARED((NN, DPAD), _f32),
            pltpu.VMEM_SHARED((NN, DPAD), _f32),
            pltpu.SemaphoreType.DMA,
            pltpu.SemaphoreType.DMA,
        ],
    )
    def k(p_hbm, q_hbm, s3_hbm, d3_hbm, w9_hbm, b9_hbm, out_hbm,
          s3v, d3v, pr_a, qr_a, pr_b, qr_b, w9_v, b9_v, ob_a, ob_b,
          psp, qsp, sem_a, sem_b):
        core = lax.axis_index("c")
        sub = lax.axis_index("s")
        wid = sub * 2 + core
        i16 = lax.iota(jnp.int32, 16)

        pltpu.sync_copy(s3_hbm.at[wid], s3v)
        pltpu.sync_copy(d3_hbm.at[wid], d3v)
        pltpu.sync_copy(w9_hbm, w9_v)
        pltpu.sync_copy(b9_hbm, b9_v)
        nslice = pl.ds(sub * NPT, NPT)
        pltpu.sync_copy(p_hbm.at[nslice], psp.at[nslice])
        pltpu.sync_copy(q_hbm.at[nslice], qsp.at[nslice])
        plsc.subcore_barrier()

        w9cols = [w9_v[pl.ds(c * 16, 16)] for c in range(4)]
        b9all = b9_v[...]
        w9b = [[jnp.broadcast_to(w9cols[c][f], (16,)) for c in range(4)]
               for f in range(10)]
        b9b = [jnp.broadcast_to(b9all[c], (16,)) for c in range(4)]
        obase = wid * EPT * 4

        def gather_pq(j, pr_v, qr_v, sem):
            pltpu.async_copy(psp.at[d3v.at[j]], pr_v, sem)
            pltpu.async_copy(qsp.at[s3v.at[j]], qr_v, sem)

        def do_chunk(j, pr_v, qr_v, ob_v, sem):
            pltpu.make_async_copy(psp.at[d3v.at[0]], pr_v, sem).wait()
            pltpu.make_async_copy(qsp.at[s3v.at[0]], qr_v, sem).wait()
            for g in range(8):
                row16 = g * 16 + i16
                o = [b9b[c] for c in range(4)]
                for f in range(10):
                    colf = jnp.full((16,), f, jnp.int32)
                    pv = plsc.load_gather(pr_v, [row16, colf])
                    qv = plsc.load_gather(qr_v, [row16, colf])
                    u = jnp.maximum(pv + qv, 0.0)
                    o = [o[c] + u * w9b[f][c] for c in range(4)]
                o = [jnp.maximum(oc, 0.0) for oc in o]
                m = jnp.maximum(jnp.maximum(o[0], o[1]), jnp.maximum(o[2], o[3]))
                ev = [jnp.exp(oc - m) for oc in o]
                r = 1.0 / (ev[0] + ev[1] + ev[2] + ev[3])
                le4 = g * 64 + i16 * 4
                for c in range(4):
                    plsc.store_scatter(ob_v, [le4 + c], ev[c] * r)
            pltpu.sync_copy(ob_v, out_hbm.at[pl.ds(obase + j * 512, 512)])

        gather_pq(0, pr_a, qr_a, sem_a)

        def pair_body(jj, carry):
            gather_pq(jj + 1, pr_b, qr_b, sem_b)
            do_chunk(jj, pr_a, qr_a, ob_a, sem_a)

            @pl.when(jj + 2 < NCH)
            def _():
                gather_pq(jj + 2, pr_a, qr_a, sem_a)

            do_chunk(jj + 1, pr_b, qr_b, ob_b, sem_b)
            return carry

        lax.fori_loop(0, NCH // 2, lambda i, c: pair_body(i * 2, c), 0)

    return k


_gat = _make_gat_edge_kernel()
_head = _make_edge_head_kernel()


def _padw(w, r, c):
    return jnp.zeros((r, c), _f32).at[:w.shape[0], :w.shape[1]].set(w)


def _padv(v, r):
    return jnp.zeros((r,), _f32).at[:v.shape[0]].set(v)


def kernel(x, e, W1, a1s, a1d, b1, W2, a2s, a2d, b2, W3, a3s, a3d, b3, We, be,
           W9, b9, edge_index):
    si = jnp.arange(N, dtype=jnp.int32)
    s2_full = jnp.zeros((EPAD2,), jnp.int32).at[:E].set(edge_index[0]).at[E:E2].set(si)
    d2_full = jnp.zeros((EPAD2,), jnp.int32).at[:E].set(edge_index[1]).at[E:E2].set(si)
    sa3 = s2_full.reshape(NW, NCH2, 128)
    da3 = d2_full.reshape(NW, NCH2, 128)

    s_flat = jnp.zeros((EPAD,), jnp.int32).at[:E].set(edge_index[0])
    d_flat = jnp.zeros((EPAD,), jnp.int32).at[:E].set(edge_index[1])
    s3 = s_flat.reshape(NW, NCH, 128)
    d3 = d_flat.reshape(NW, NCH, 128)

    hp, hs, hd, c = _tc_prep_x(
        x, _padw(W1, 128, DPAD), _padw(a1s[:, None], DPAD, 1),
        _padw(a1d[:, None], DPAD, 1), 5)
    acc = _gat(hp, hs.reshape(NN), hd.reshape(NN), c.reshape(NN), sa3, da3)

    hp, hs, hd, c = _tc_prep_acc(
        acc, _padv(b1, DPAD)[None, :], _padw(W2, DPAD, DPAD),
        _padw(a2s[:, None], DPAD, 1), _padw(a2d[:, None], DPAD, 1), 5, 10)
    acc = _gat(hp, hs.reshape(NN), hd.reshape(NN), c.reshape(NN), sa3, da3)

    hp, hs, hd, c = _tc_prep_acc(
        acc, _padv(b2, DPAD)[None, :], _padw(W3, DPAD, DPAD),
        _padw(a3s[:, None], DPAD, 1), _padw(a3d[:, None], DPAD, 1), 10, 10)
    acc = _gat(hp, hs.reshape(NN), hd.reshape(NN), c.reshape(NN), sa3, da3)

    p, q = _tc_prep_final(
        acc, _padv(b3, DPAD)[None, :], _padw(We[:10] - We[10:], DPAD, DPAD),
        _padw(We[10:], DPAD, DPAD), _padv(be, DPAD)[None, :], 10)
    w9cols = jnp.zeros((64,), _f32).at[0:10].set(W9[:, 0]).at[16:26].set(
        W9[:, 1]).at[32:42].set(W9[:, 2]).at[48:58].set(W9[:, 3])
    out = _head(p, q, s3, d3, w9cols, _padv(b9, 16))
    return out.reshape(EPAD, 4)[:E]


# speedup vs baseline: 1.3904x; 1.0530x over previous
import functools

import jax
import jax.numpy as jnp
from jax import lax
from jax.experimental import pallas as pl
from jax.experimental.pallas import tpu as pltpu
from jax.experimental.pallas import tpu_sc as plsc

N = 10000
E = 320000
DPAD = 16
NW = 32
NN = 10240
NPT = NN // 16

E2 = E + N
NCH2 = 82
EPT2 = NCH2 * 128
EPAD2 = NW * EPT2

NCH = 80
EPT = NCH * 128
EPAD = NW * EPT

_f32 = jnp.float32


def _emit_node_tables(h, as_ref, ad_ref, hp_ref, hs_ref, hd_ref, c_ref, d_out):
    col = lax.broadcasted_iota(jnp.int32, (N, DPAD), 1)
    hp_ref[:N, :] = h + jnp.where(col == d_out, 1.0, 0.0).astype(_f32)
    hp_ref[N:, :] = jnp.zeros((NN - N, DPAD), _f32)
    hs = jnp.dot(h, as_ref[...], preferred_element_type=_f32)
    hd = jnp.dot(h, ad_ref[...], preferred_element_type=_f32)
    hs_ref[:N, :] = hs
    hs_ref[N:, :] = jnp.zeros((NN - N, 1), _f32)
    hd_ref[:N, :] = hd
    hd_ref[N:, :] = jnp.zeros((NN - N, 1), _f32)
    stab = jnp.max(hs) + hd
    c_ref[:N, :] = jnp.maximum(stab, 0.2 * stab)
    c_ref[N:, :] = jnp.zeros((NN - N, 1), _f32)


def _prep_from_x(x_ref, w_ref, as_ref, ad_ref, hp_ref, hs_ref, hd_ref, c_ref, *, d_out):
    h = jnp.dot(x_ref[...], w_ref[...], preferred_element_type=_f32)
    _emit_node_tables(h, as_ref, ad_ref, hp_ref, hs_ref, hd_ref, c_ref, d_out)


def _prep_from_acc(acc_ref, b_ref, w_ref, as_ref, ad_ref, hp_ref, hs_ref, hd_ref,
                   c_ref, *, d_prev, d_out):
    num = acc_ref[:N, :] + acc_ref[NN:NN + N, :]
    den = num[:, d_prev:d_prev + 1] + 1e-16
    hprev = jnp.maximum(num / den + b_ref[...], 0.0)
    h = jnp.dot(hprev, w_ref[...], preferred_element_type=_f32)
    _emit_node_tables(h, as_ref, ad_ref, hp_ref, hs_ref, hd_ref, c_ref, d_out)


def _prep_final(acc_ref, b_ref, wa_ref, wb_ref, be_ref, p_ref, q_ref, *, d_prev):
    num = acc_ref[:N, :] + acc_ref[NN:NN + N, :]
    den = num[:, d_prev:d_prev + 1] + 1e-16
    h = jnp.maximum(num / den + b_ref[...], 0.0)
    p_ref[:N, :] = jnp.dot(h, wa_ref[...], preferred_element_type=_f32) + be_ref[...]
    p_ref[N:, :] = jnp.zeros((NN - N, DPAD), _f32)
    q_ref[:N, :] = jnp.dot(h, wb_ref[...], preferred_element_type=_f32)
    q_ref[N:, :] = jnp.zeros((NN - N, DPAD), _f32)


_TABLE_OUT = [
    jax.ShapeDtypeStruct((NN, DPAD), _f32),
    jax.ShapeDtypeStruct((NN, 1), _f32),
    jax.ShapeDtypeStruct((NN, 1), _f32),
    jax.ShapeDtypeStruct((NN, 1), _f32),
]


def _tc_prep_x(x, wp, asp, adp, d_out):
    return pl.pallas_call(
        functools.partial(_prep_from_x, d_out=d_out),
        out_shape=_TABLE_OUT,
    )(x, wp, asp, adp)


def _tc_prep_acc(acc, bp, wp, asp, adp, d_prev, d_out):
    return pl.pallas_call(
        functools.partial(_prep_from_acc, d_prev=d_prev, d_out=d_out),
        out_shape=_TABLE_OUT,
    )(acc, bp, wp, asp, adp)


def _tc_prep_final(acc, bp, wap, wbp, bep, d_prev):
    return pl.pallas_call(
        functools.partial(_prep_final, d_prev=d_prev),
        out_shape=[
            jax.ShapeDtypeStruct((NN, DPAD), _f32),
            jax.ShapeDtypeStruct((NN, DPAD), _f32),
        ],
    )(acc, bp, wap, wbp, bep)


def _make_gat_edge_kernel():
    mesh = plsc.VectorSubcoreMesh(core_axis_name="c", subcore_axis_name="s")

    @functools.partial(
        pl.kernel, mesh=mesh,
        compiler_params=pltpu.CompilerParams(
            needs_layout_passes=False, use_tc_tiling_on_sc=False),
        out_type=jax.ShapeDtypeStruct((2 * NN, DPAD), _f32),
        scratch_types=[
            pltpu.VMEM((NN,), _f32),
            pltpu.VMEM((NN,), _f32),
            pltpu.VMEM((NN,), _f32),
            pltpu.VMEM((NCH2, 128), jnp.int32),
            pltpu.VMEM((NCH2, 128), jnp.int32),
            pltpu.VMEM((128, DPAD), _f32),
            pltpu.VMEM((128, DPAD), _f32),
            pltpu.VMEM((NPT, DPAD), _f32),
            pltpu.VMEM_SHARED((NN, DPAD), _f32),
            pltpu.VMEM_SHARED((NN, DPAD), _f32),
            pltpu.SemaphoreType.DMA,
            pltpu.SemaphoreType.DMA,
        ],
    )
    def k(hp_hbm, hs_hbm, hd_hbm, c_hbm, s3_hbm, d3_hbm, out_hbm,
          hs_v, hd_v, c_v, s3v, d3v, rows_a, rows_b, z_v, hsp, accsp,
          sem_a, sem_b):
        core = lax.axis_index("c")
        sub = lax.axis_index("s")
        wid = sub * 2 + core
        i16 = lax.iota(jnp.int32, 16)
        zero16 = jnp.zeros((16,), _f32)

        pltpu.sync_copy(hs_hbm, hs_v)
        pltpu.sync_copy(hd_hbm, hd_v)
        pltpu.sync_copy(c_hbm, c_v)
        pltpu.sync_copy(s3_hbm.at[wid], s3v)
        pltpu.sync_copy(d3_hbm.at[wid], d3v)
        nslice = pl.ds(sub * NPT, NPT)
        pltpu.sync_copy(hp_hbm.at[nslice], hsp.at[nslice])
        for r in range(NPT):
            z_v[r, :] = zero16
        pltpu.sync_copy(z_v, accsp.at[nslice])
        plsc.subcore_barrier()

        ebase = wid * EPT2
        bufs = (rows_a, rows_b)
        sems = (sem_a, sem_b)

        def do_chunk(j, rows_v, sem):
            pltpu.make_async_copy(hsp.at[s3v.at[j]], rows_v, sem).wait()
            for g in range(8):
                s16 = s3v[j, pl.ds(g * 16, 16)]
                d16 = d3v[j, pl.ds(g * 16, 16)]
                hs_g = plsc.load_gather(hs_v, [s16])
                hd_g = plsc.load_gather(hd_v, [d16])
                c_g = plsc.load_gather(c_v, [d16])
                z = hs_g + hd_g
                lg = jnp.maximum(z, 0.2 * z)
                ex = jnp.exp(lg - c_g)
                eid = ebase + j * 128 + g * 16 + i16
                ex = jnp.where(eid < E2, ex, 0.0)
                for kk in range(16):
                    r = g * 16 + kk
                    exk = jnp.broadcast_to(ex[kk], (16,))
                    rows_v[r, :] = rows_v[r, :] * exk
            pltpu.sync_copy(rows_v, accsp.at[d3v.at[j]], add=True)

        pltpu.async_copy(hsp.at[s3v.at[0]], rows_a, sem_a)

        def pair_body(jj, carry):
            pltpu.async_copy(hsp.at[s3v.at[jj + 1]], rows_b, sem_b)
            do_chunk(jj, rows_a, sem_a)

            @pl.when(jj + 2 < NCH2)
            def _():
                pltpu.async_copy(hsp.at[s3v.at[jj + 2]], rows_a, sem_a)

            do_chunk(jj + 1, rows_b, sem_b)
            return carry

        lax.fori_loop(0, NCH2 // 2, lambda i, c: pair_body(i * 2, c), 0)
        plsc.subcore_barrier()
        pltpu.sync_copy(accsp.at[nslice],
                        out_hbm.at[pl.ds(core * NN + sub * NPT, NPT)])

    return k


def _make_edge_head_kernel():
    mesh = plsc.VectorSubcoreMesh(core_axis_name="c", subcore_axis_name="s")

    @functools.partial(
        pl.kernel, mesh=mesh,
        compiler_params=pltpu.CompilerParams(
            needs_layout_passes=False, use_tc_tiling_on_sc=False),
        out_type=jax.ShapeDtypeStruct((EPAD, DPAD), _f32),
        scratch_types=[
            pltpu.VMEM((NCH, 128), jnp.int32),
            pltpu.VMEM((NCH, 128), jnp.int32),
            pltpu.VMEM((128, DPAD), _f32),
            pltpu.VMEM((128, DPAD), _f32),
            pltpu.VMEM((128, DPAD), _f32),
            pltpu.VMEM((128, DPAD), _f32),
            pltpu.VMEM_SHARED((NN, DPAD), _f32),
            pltpu.VMEM_SHARED((NN, DPAD), _f32),
            pltpu.SemaphoreType.DMA,
            pltpu.SemaphoreType.DMA,
        ],
    )
    def k(p_hbm, q_hbm, s3_hbm, d3_hbm, out_hbm,
          s3v, d3v, pr_a, qr_a, pr_b, qr_b, psp, qsp, sem_a, sem_b):
        core = lax.axis_index("c")
        sub = lax.axis_index("s")
        wid = sub * 2 + core

        pltpu.sync_copy(s3_hbm.at[wid], s3v)
        pltpu.sync_copy(d3_hbm.at[wid], d3v)
        nslice = pl.ds(sub * NPT, NPT)
        pltpu.sync_copy(p_hbm.at[nslice], psp.at[nslice])
        pltpu.sync_copy(q_hbm.at[nslice], qsp.at[nslice])
        plsc.subcore_barrier()

        ebase = wid * EPT

        def gather_pq(j, pr_v, qr_v, sem):
            pltpu.async_copy(psp.at[d3v.at[j]], pr_v, sem)
            pltpu.async_copy(qsp.at[s3v.at[j]], qr_v, sem)

        def do_chunk(j, pr_v, qr_v, sem):
            pltpu.make_async_copy(psp.at[d3v.at[0]], pr_v, sem).wait()
            pltpu.make_async_copy(qsp.at[s3v.at[0]], qr_v, sem).wait()
            for r in range(128):
                pr_v[r, :] = jnp.maximum(pr_v[r, :] + qr_v[r, :], 0.0)
            pltpu.sync_copy(pr_v, out_hbm.at[pl.ds(ebase + j * 128, 128)])

        gather_pq(0, pr_a, qr_a, sem_a)

        def pair_body(jj, carry):
            gather_pq(jj + 1, pr_b, qr_b, sem_b)
            do_chunk(jj, pr_a, qr_a, sem_a)

            @pl.when(jj + 2 < NCH)
            def _():
                gather_pq(jj + 2, pr_a, qr_a, sem_a)

            do_chunk(jj + 1, pr_b, qr_b, sem_b)
            return carry

        lax.fori_loop(0, NCH // 2, lambda i, c: pair_body(i * 2, c), 0)

    return k


_HEAD_BLK = 4096


def _head_epilogue(u_ref, w_ref, b_ref, out_ref):
    o = jnp.dot(u_ref[...], w_ref[...], preferred_element_type=_f32) + b_ref[...]
    o = jnp.maximum(o, 0.0)
    m = jnp.max(o, axis=1, keepdims=True)
    ev = jnp.exp(o - m)
    out_ref[...] = ev / jnp.sum(ev, axis=1, keepdims=True)


def _tc_head(u, w9p, b9p):
    return pl.pallas_call(
        _head_epilogue,
        grid=(EPAD // _HEAD_BLK,),
        in_specs=[
            pl.BlockSpec((_HEAD_BLK, DPAD), lambda i: (i, 0)),
            pl.BlockSpec((DPAD, 4), lambda i: (0, 0)),
            pl.BlockSpec((1, 4), lambda i: (0, 0)),
        ],
        out_specs=pl.BlockSpec((_HEAD_BLK, 4), lambda i: (i, 0)),
        out_shape=jax.ShapeDtypeStruct((EPAD, 4), _f32),
    )(u, w9p, b9p)


_gat = _make_gat_edge_kernel()
_head = _make_edge_head_kernel()


def _padw(w, r, c):
    return jnp.zeros((r, c), _f32).at[:w.shape[0], :w.shape[1]].set(w)


def _padv(v, r):
    return jnp.zeros((r,), _f32).at[:v.shape[0]].set(v)


def kernel(x, e, W1, a1s, a1d, b1, W2, a2s, a2d, b2, W3, a3s, a3d, b3, We, be,
           W9, b9, edge_index):
    si = jnp.arange(N, dtype=jnp.int32)
    s2_full = jnp.zeros((EPAD2,), jnp.int32).at[:E].set(edge_index[0]).at[E:E2].set(si)
    d2_full = jnp.zeros((EPAD2,), jnp.int32).at[:E].set(edge_index[1]).at[E:E2].set(si)
    sa3 = s2_full.reshape(NW, NCH2, 128)
    da3 = d2_full.reshape(NW, NCH2, 128)

    s_flat = jnp.zeros((EPAD,), jnp.int32).at[:E].set(edge_index[0])
    d_flat = jnp.zeros((EPAD,), jnp.int32).at[:E].set(edge_index[1])
    s3 = s_flat.reshape(NW, NCH, 128)
    d3 = d_flat.reshape(NW, NCH, 128)

    hp, hs, hd, c = _tc_prep_x(
        x, _padw(W1, 128, DPAD), _padw(a1s[:, None], DPAD, 1),
        _padw(a1d[:, None], DPAD, 1), 5)
    acc = _gat(hp, hs.reshape(NN), hd.reshape(NN), c.reshape(NN), sa3, da3)

    hp, hs, hd, c = _tc_prep_acc(
        acc, _padv(b1, DPAD)[None, :], _padw(W2, DPAD, DPAD),
        _padw(a2s[:, None], DPAD, 1), _padw(a2d[:, None], DPAD, 1), 5, 10)
    acc = _gat(hp, hs.reshape(NN), hd.reshape(NN), c.reshape(NN), sa3, da3)

    hp, hs, hd, c = _tc_prep_acc(
        acc, _padv(b2, DPAD)[None, :], _padw(W3, DPAD, DPAD),
        _padw(a3s[:, None], DPAD, 1), _padw(a3d[:, None], DPAD, 1), 10, 10)
    acc = _gat(hp, hs.reshape(NN), hd.reshape(NN), c.reshape(NN), sa3, da3)

    p, q = _tc_prep_final(
        acc, _padv(b3, DPAD)[None, :], _padw(We[:10] - We[10:], DPAD, DPAD),
        _padw(We[10:], DPAD, DPAD), _padv(be, DPAD)[None, :], 10)
    u = _head(p, q, s3, d3)
    out = _tc_head(u, _padw(W9, DPAD, 4), b9[None, :])
    return out[:E]


# speedup vs baseline: 1.6283x; 1.1711x over previous
import functools

import jax
import jax.numpy as jnp
from jax import lax
from jax.experimental import pallas as pl
from jax.experimental.pallas import tpu as pltpu
from jax.experimental.pallas import tpu_sc as plsc

N = 10000
E = 320000
DPAD = 16
NW = 32
NN = 10240
NPT = NN // 16

E2 = E + N
NCH2 = 82
EPT2 = NCH2 * 128
EPAD2 = NW * EPT2

NCH = 80
EPT = NCH * 128
EPAD = NW * EPT

_f32 = jnp.float32


def _emit_node_tables(h, as_ref, ad_ref, hp_ref, hs_ref, hd_ref, c_ref, d_out):
    col = lax.broadcasted_iota(jnp.int32, (N, DPAD), 1)
    hp_ref[:N, :] = h + jnp.where(col == d_out, 1.0, 0.0).astype(_f32)
    hp_ref[N:, :] = jnp.zeros((NN - N, DPAD), _f32)
    hs = jnp.dot(h, as_ref[...], preferred_element_type=_f32)
    hd = jnp.dot(h, ad_ref[...], preferred_element_type=_f32)
    hs_ref[:N, :] = hs
    hs_ref[N:, :] = jnp.zeros((NN - N, 1), _f32)
    hd_ref[:N, :] = hd
    hd_ref[N:, :] = jnp.zeros((NN - N, 1), _f32)
    stab = jnp.max(hs) + hd
    c_ref[:N, :] = jnp.maximum(stab, 0.2 * stab)
    c_ref[N:, :] = jnp.zeros((NN - N, 1), _f32)


def _prep_from_x(x_ref, w_ref, as_ref, ad_ref, hp_ref, hs_ref, hd_ref, c_ref, *, d_out):
    h = jnp.dot(x_ref[...], w_ref[...], preferred_element_type=_f32)
    _emit_node_tables(h, as_ref, ad_ref, hp_ref, hs_ref, hd_ref, c_ref, d_out)


def _prep_from_acc(acc_ref, b_ref, w_ref, as_ref, ad_ref, hp_ref, hs_ref, hd_ref,
                   c_ref, *, d_prev, d_out):
    num = acc_ref[:N, :] + acc_ref[NN:NN + N, :]
    den = num[:, d_prev:d_prev + 1] + 1e-16
    hprev = jnp.maximum(num / den + b_ref[...], 0.0)
    h = jnp.dot(hprev, w_ref[...], preferred_element_type=_f32)
    _emit_node_tables(h, as_ref, ad_ref, hp_ref, hs_ref, hd_ref, c_ref, d_out)


def _prep_final(acc_ref, b_ref, wa_ref, wb_ref, be_ref, p_ref, q_ref, *, d_prev):
    num = acc_ref[:N, :] + acc_ref[NN:NN + N, :]
    den = num[:, d_prev:d_prev + 1] + 1e-16
    h = jnp.maximum(num / den + b_ref[...], 0.0)
    p_ref[:N, :] = jnp.dot(h, wa_ref[...], preferred_element_type=_f32) + be_ref[...]
    p_ref[N:, :] = jnp.zeros((NN - N, DPAD), _f32)
    q_ref[:N, :] = jnp.dot(h, wb_ref[...], preferred_element_type=_f32)
    q_ref[N:, :] = jnp.zeros((NN - N, DPAD), _f32)


_TABLE_OUT = [
    jax.ShapeDtypeStruct((NN, DPAD), _f32),
    jax.ShapeDtypeStruct((NN, 1), _f32),
    jax.ShapeDtypeStruct((NN, 1), _f32),
    jax.ShapeDtypeStruct((NN, 1), _f32),
]


def _tc_prep_x(x, wp, asp, adp, d_out):
    return pl.pallas_call(
        functools.partial(_prep_from_x, d_out=d_out),
        out_shape=_TABLE_OUT,
    )(x, wp, asp, adp)


def _tc_prep_acc(acc, bp, wp, asp, adp, d_prev, d_out):
    return pl.pallas_call(
        functools.partial(_prep_from_acc, d_prev=d_prev, d_out=d_out),
        out_shape=_TABLE_OUT,
    )(acc, bp, wp, asp, adp)


def _tc_prep_final(acc, bp, wap, wbp, bep, d_prev):
    return pl.pallas_call(
        functools.partial(_prep_final, d_prev=d_prev),
        out_shape=[
            jax.ShapeDtypeStruct((NN, DPAD), _f32),
            jax.ShapeDtypeStruct((NN, DPAD), _f32),
        ],
    )(acc, bp, wap, wbp, bep)


def _make_gat_edge_kernel():
    mesh = plsc.VectorSubcoreMesh(core_axis_name="c", subcore_axis_name="s")

    @functools.partial(
        pl.kernel, mesh=mesh,
        compiler_params=pltpu.CompilerParams(
            needs_layout_passes=False, use_tc_tiling_on_sc=False),
        out_type=jax.ShapeDtypeStruct((2 * NN, DPAD), _f32),
        scratch_types=[
            pltpu.VMEM((NN,), _f32),
            pltpu.VMEM((NN,), _f32),
            pltpu.VMEM((NN,), _f32),
            pltpu.VMEM((NCH2, 128), jnp.int32),
            pltpu.VMEM((NCH2, 128), jnp.int32),
            pltpu.VMEM((128, DPAD), _f32),
            pltpu.VMEM((128, DPAD), _f32),
            pltpu.VMEM((NPT, DPAD), _f32),
            pltpu.VMEM_SHARED((NN, DPAD), _f32),
            pltpu.VMEM_SHARED((NN, DPAD), _f32),
            pltpu.SemaphoreType.DMA,
            pltpu.SemaphoreType.DMA,
        ],
    )
    def k(hp_hbm, hs_hbm, hd_hbm, c_hbm, s3_hbm, d3_hbm, out_hbm,
          hs_v, hd_v, c_v, s3v, d3v, rows_a, rows_b, z_v, hsp, accsp,
          sem_a, sem_b):
        core = lax.axis_index("c")
        sub = lax.axis_index("s")
        wid = sub * 2 + core
        i16 = lax.iota(jnp.int32, 16)
        zero16 = jnp.zeros((16,), _f32)

        pltpu.sync_copy(hs_hbm, hs_v)
        pltpu.sync_copy(hd_hbm, hd_v)
        pltpu.sync_copy(c_hbm, c_v)
        pltpu.sync_copy(s3_hbm.at[wid], s3v)
        pltpu.sync_copy(d3_hbm.at[wid], d3v)
        nslice = pl.ds(sub * NPT, NPT)
        pltpu.sync_copy(hp_hbm.at[nslice], hsp.at[nslice])
        for r in range(NPT):
            z_v[r, :] = zero16
        pltpu.sync_copy(z_v, accsp.at[nslice])
        plsc.subcore_barrier()

        ebase = wid * EPT2
        bufs = (rows_a, rows_b)
        sems = (sem_a, sem_b)

        def do_chunk(j, rows_v, sem):
            pltpu.make_async_copy(hsp.at[s3v.at[j]], rows_v, sem).wait()
            for g in range(8):
                s16 = s3v[j, pl.ds(g * 16, 16)]
                d16 = d3v[j, pl.ds(g * 16, 16)]
                hs_g = plsc.load_gather(hs_v, [s16])
                hd_g = plsc.load_gather(hd_v, [d16])
                c_g = plsc.load_gather(c_v, [d16])
                z = hs_g + hd_g
                lg = jnp.maximum(z, 0.2 * z)
                ex = jnp.exp(lg - c_g)
                eid = ebase + j * 128 + g * 16 + i16
                ex = jnp.where(eid < E2, ex, 0.0)
                for kk in range(16):
                    r = g * 16 + kk
                    exk = jnp.broadcast_to(ex[kk], (16,))
                    rows_v[r, :] = rows_v[r, :] * exk
            pltpu.sync_copy(rows_v, accsp.at[d3v.at[j]], add=True)

        pltpu.async_copy(hsp.at[s3v.at[0]], rows_a, sem_a)

        def pair_body(jj, carry):
            pltpu.async_copy(hsp.at[s3v.at[jj + 1]], rows_b, sem_b)
            do_chunk(jj, rows_a, sem_a)

            @pl.when(jj + 2 < NCH2)
            def _():
                pltpu.async_copy(hsp.at[s3v.at[jj + 2]], rows_a, sem_a)

            do_chunk(jj + 1, rows_b, sem_b)
            return carry

        lax.fori_loop(0, NCH2 // 2, lambda i, c: pair_body(i * 2, c), 0)
        plsc.subcore_barrier()
        pltpu.sync_copy(accsp.at[nslice],
                        out_hbm.at[pl.ds(core * NN + sub * NPT, NPT)])

    return k


def _make_edge_head_kernel():
    mesh = plsc.VectorSubcoreMesh(core_axis_name="c", subcore_axis_name="s")

    @functools.partial(
        pl.kernel, mesh=mesh,
        compiler_params=pltpu.CompilerParams(
            needs_layout_passes=False, use_tc_tiling_on_sc=False),
        out_type=jax.ShapeDtypeStruct((EPAD * 4,), _f32),
        scratch_types=[
            pltpu.VMEM((NCH, 128), jnp.int32),
            pltpu.VMEM((NCH, 128), jnp.int32),
            pltpu.VMEM((128, DPAD), _f32),
            pltpu.VMEM((128, DPAD), _f32),
            pltpu.VMEM((128, DPAD), _f32),
            pltpu.VMEM((128, DPAD), _f32),
            pltpu.VMEM((128 * 17,), _f32),
            pltpu.VMEM((64,), _f32),
            pltpu.VMEM((16,), _f32),
            pltpu.VMEM((512,), _f32),
            pltpu.VMEM((512,), _f32),
            pltpu.VMEM_SHARED((NN, DPAD), _f32),
            pltpu.VMEM_SHARED((NN, DPAD), _f32),
            pltpu.SemaphoreType.DMA,
            pltpu.SemaphoreType.DMA,
        ],
    )
    def k(p_hbm, q_hbm, s3_hbm, d3_hbm, w9_hbm, b9_hbm, out_hbm,
          s3v, d3v, pr_a, qr_a, pr_b, qr_b, ut_v, w9_v, b9_v, ob_a, ob_b,
          psp, qsp, sem_a, sem_b):
        core = lax.axis_index("c")
        sub = lax.axis_index("s")
        wid = sub * 2 + core
        i16 = lax.iota(jnp.int32, 16)

        pltpu.sync_copy(s3_hbm.at[wid], s3v)
        pltpu.sync_copy(d3_hbm.at[wid], d3v)
        pltpu.sync_copy(w9_hbm, w9_v)
        pltpu.sync_copy(b9_hbm, b9_v)
        nslice = pl.ds(sub * NPT, NPT)
        pltpu.sync_copy(p_hbm.at[nslice], psp.at[nslice])
        pltpu.sync_copy(q_hbm.at[nslice], qsp.at[nslice])
        plsc.subcore_barrier()

        w9cols = [w9_v[pl.ds(c * 16, 16)] for c in range(4)]
        b9all = b9_v[...]
        w9b = [[jnp.broadcast_to(w9cols[c][f], (16,)) for c in range(4)]
               for f in range(10)]
        b9b = [jnp.broadcast_to(b9all[c], (16,)) for c in range(4)]
        i16x17 = i16 * 17
        i16x4 = i16 * 4
        obase = wid * EPT * 4

        def gather_pq(j, pr_v, qr_v, sem):
            pltpu.async_copy(psp.at[d3v.at[j]], pr_v, sem)
            pltpu.async_copy(qsp.at[s3v.at[j]], qr_v, sem)

        def do_chunk(j, pr_v, qr_v, ob_v, sem):
            pltpu.make_async_copy(psp.at[d3v.at[0]], pr_v, sem).wait()
            pltpu.make_async_copy(qsp.at[s3v.at[0]], qr_v, sem).wait()
            for r in range(128):
                u = jnp.maximum(pr_v[r, :] + qr_v[r, :], 0.0)
                ut_v[pl.ds(r * 17, 16)] = u
            for g in range(8):
                o = [b9b[c] for c in range(4)]
                base17 = g * 16 * 17
                for f in range(10):
                    uf = plsc.load_gather(ut_v, [base17 + i16x17 + f])
                    o = [o[c] + uf * w9b[f][c] for c in range(4)]
                o = [jnp.maximum(oc, 0.0) for oc in o]
                m = jnp.maximum(jnp.maximum(o[0], o[1]), jnp.maximum(o[2], o[3]))
                ev = [jnp.exp(oc - m) for oc in o]
                r = 1.0 / (ev[0] + ev[1] + ev[2] + ev[3])
                le4 = g * 64 + i16x4
                for c in range(4):
                    plsc.store_scatter(ob_v, [le4 + c], ev[c] * r)
            pltpu.sync_copy(ob_v, out_hbm.at[pl.ds(obase + j * 512, 512)])

        gather_pq(0, pr_a, qr_a, sem_a)

        def pair_body(jj, carry):
            gather_pq(jj + 1, pr_b, qr_b, sem_b)
            do_chunk(jj, pr_a, qr_a, ob_a, sem_a)

            @pl.when(jj + 2 < NCH)
            def _():
                gather_pq(jj + 2, pr_a, qr_a, sem_a)

            do_chunk(jj + 1, pr_b, qr_b, ob_b, sem_b)
            return carry

        lax.fori_loop(0, NCH // 2, lambda i, c: pair_body(i * 2, c), 0)

    return k


_gat = _make_gat_edge_kernel()
_head = _make_edge_head_kernel()


def _padw(w, r, c):
    return jnp.zeros((r, c), _f32).at[:w.shape[0], :w.shape[1]].set(w)


def _padv(v, r):
    return jnp.zeros((r,), _f32).at[:v.shape[0]].set(v)


def kernel(x, e, W1, a1s, a1d, b1, W2, a2s, a2d, b2, W3, a3s, a3d, b3, We, be,
           W9, b9, edge_index):
    si = jnp.arange(N, dtype=jnp.int32)
    s2_full = jnp.zeros((EPAD2,), jnp.int32).at[:E].set(edge_index[0]).at[E:E2].set(si)
    d2_full = jnp.zeros((EPAD2,), jnp.int32).at[:E].set(edge_index[1]).at[E:E2].set(si)
    sa3 = s2_full.reshape(NW, NCH2, 128)
    da3 = d2_full.reshape(NW, NCH2, 128)

    s_flat = jnp.zeros((EPAD,), jnp.int32).at[:E].set(edge_index[0])
    d_flat = jnp.zeros((EPAD,), jnp.int32).at[:E].set(edge_index[1])
    s3 = s_flat.reshape(NW, NCH, 128)
    d3 = d_flat.reshape(NW, NCH, 128)

    hp, hs, hd, c = _tc_prep_x(
        x, _padw(W1, 128, DPAD), _padw(a1s[:, None], DPAD, 1),
        _padw(a1d[:, None], DPAD, 1), 5)
    acc = _gat(hp, hs.reshape(NN), hd.reshape(NN), c.reshape(NN), sa3, da3)

    hp, hs, hd, c = _tc_prep_acc(
        acc, _padv(b1, DPAD)[None, :], _padw(W2, DPAD, DPAD),
        _padw(a2s[:, None], DPAD, 1), _padw(a2d[:, None], DPAD, 1), 5, 10)
    acc = _gat(hp, hs.reshape(NN), hd.reshape(NN), c.reshape(NN), sa3, da3)

    hp, hs, hd, c = _tc_prep_acc(
        acc, _padv(b2, DPAD)[None, :], _padw(W3, DPAD, DPAD),
        _padw(a3s[:, None], DPAD, 1), _padw(a3d[:, None], DPAD, 1), 10, 10)
    acc = _gat(hp, hs.reshape(NN), hd.reshape(NN), c.reshape(NN), sa3, da3)

    p, q = _tc_prep_final(
        acc, _padv(b3, DPAD)[None, :], _padw(We[:10] - We[10:], DPAD, DPAD),
        _padw(We[10:], DPAD, DPAD), _padv(be, DPAD)[None, :], 10)
    w9cols = jnp.zeros((64,), _f32).at[0:10].set(W9[:, 0]).at[16:26].set(
        W9[:, 1]).at[32:42].set(W9[:, 2]).at[48:58].set(W9[:, 3])
    out = _head(p, q, s3, d3, w9cols, _padv(b9, 16))
    return out.reshape(EPAD, 4)[:E]


# speedup vs baseline: 1.6896x; 1.0376x over previous
import functools

import jax
import jax.numpy as jnp
from jax import lax
from jax.experimental import pallas as pl
from jax.experimental.pallas import tpu as pltpu
from jax.experimental.pallas import tpu_sc as plsc

N = 10000
E = 320000
DPAD = 16
NW = 32
NN = 10240
NPT = NN // 16

E2 = E + N
NCH2 = 82
EPT2 = NCH2 * 128
EPAD2 = NW * EPT2

NCH = 80
EPT = NCH * 128
EPAD = NW * EPT

_f32 = jnp.float32


def _emit_node_tables(h, as_ref, ad_ref, hp_ref, hs_ref, hdp_ref, ms_ref, d_out):
    col = lax.broadcasted_iota(jnp.int32, (N, DPAD), 1)
    hp_ref[:N, :] = h + jnp.where(col == d_out, 1.0, 0.0).astype(_f32)
    hp_ref[N:, :] = jnp.zeros((NN - N, DPAD), _f32)
    hs = jnp.dot(h, as_ref[...], preferred_element_type=_f32)
    hd = jnp.dot(h, ad_ref[...], preferred_element_type=_f32)
    hs_ref[:N, :] = hs
    hs_ref[N:, :] = jnp.zeros((NN - N, 1), _f32)
    maxs = jnp.max(hs)
    hdp_ref[:N, :] = maxs + hd
    hdp_ref[N:, :] = jnp.zeros((NN - N, 1), _f32)
    ms_ref[...] = jnp.broadcast_to(maxs, (1, DPAD))


def _prep_from_x(x_ref, w_ref, as_ref, ad_ref, hp_ref, hs_ref, hd_ref, c_ref, *, d_out):
    h = jnp.dot(x_ref[...], w_ref[...], preferred_element_type=_f32)
    _emit_node_tables(h, as_ref, ad_ref, hp_ref, hs_ref, hd_ref, c_ref, d_out)


def _prep_from_acc(acc_ref, b_ref, w_ref, as_ref, ad_ref, hp_ref, hs_ref, hd_ref,
                   c_ref, *, d_prev, d_out):
    num = acc_ref[:N, :] + acc_ref[NN:NN + N, :]
    den = num[:, d_prev:d_prev + 1] + 1e-16
    hprev = jnp.maximum(num / den + b_ref[...], 0.0)
    h = jnp.dot(hprev, w_ref[...], preferred_element_type=_f32)
    _emit_node_tables(h, as_ref, ad_ref, hp_ref, hs_ref, hd_ref, c_ref, d_out)


def _prep_final(acc_ref, b_ref, wa_ref, wb_ref, be_ref, p_ref, q_ref, *, d_prev):
    num = acc_ref[:N, :] + acc_ref[NN:NN + N, :]
    den = num[:, d_prev:d_prev + 1] + 1e-16
    h = jnp.maximum(num / den + b_ref[...], 0.0)
    p_ref[:N, :] = jnp.dot(h, wa_ref[...], preferred_element_type=_f32) + be_ref[...]
    p_ref[N:, :] = jnp.zeros((NN - N, DPAD), _f32)
    q_ref[:N, :] = jnp.dot(h, wb_ref[...], preferred_element_type=_f32)
    q_ref[N:, :] = jnp.zeros((NN - N, DPAD), _f32)


_TABLE_OUT = [
    jax.ShapeDtypeStruct((NN, DPAD), _f32),
    jax.ShapeDtypeStruct((NN, 1), _f32),
    jax.ShapeDtypeStruct((NN, 1), _f32),
    jax.ShapeDtypeStruct((1, DPAD), _f32),
]


def _tc_prep_x(x, wp, asp, adp, d_out):
    return pl.pallas_call(
        functools.partial(_prep_from_x, d_out=d_out),
        out_shape=_TABLE_OUT,
    )(x, wp, asp, adp)


def _tc_prep_acc(acc, bp, wp, asp, adp, d_prev, d_out):
    return pl.pallas_call(
        functools.partial(_prep_from_acc, d_prev=d_prev, d_out=d_out),
        out_shape=_TABLE_OUT,
    )(acc, bp, wp, asp, adp)


def _tc_prep_final(acc, bp, wap, wbp, bep, d_prev):
    return pl.pallas_call(
        functools.partial(_prep_final, d_prev=d_prev),
        out_shape=[
            jax.ShapeDtypeStruct((NN, DPAD), _f32),
            jax.ShapeDtypeStruct((NN, DPAD), _f32),
        ],
    )(acc, bp, wap, wbp, bep)


def _make_gat_edge_kernel():
    mesh = plsc.VectorSubcoreMesh(core_axis_name="c", subcore_axis_name="s")

    @functools.partial(
        pl.kernel, mesh=mesh,
        compiler_params=pltpu.CompilerParams(
            needs_layout_passes=False, use_tc_tiling_on_sc=False,
            skip_device_barrier=True),
        out_type=jax.ShapeDtypeStruct((2 * NN, DPAD), _f32),
        scratch_types=[
            pltpu.VMEM((NN,), _f32),
            pltpu.VMEM((NN,), _f32),
            pltpu.VMEM((16,), _f32),
            pltpu.VMEM((NCH2, 128), jnp.int32),
            pltpu.VMEM((NCH2, 128), jnp.int32),
            pltpu.VMEM((128, DPAD), _f32),
            pltpu.VMEM((128, DPAD), _f32),
            pltpu.VMEM((NPT, DPAD), _f32),
            pltpu.VMEM_SHARED((NN, DPAD), _f32),
            pltpu.VMEM_SHARED((NN, DPAD), _f32),
            pltpu.SemaphoreType.DMA,
            pltpu.SemaphoreType.DMA,
        ],
    )
    def k(hp_hbm, hs_hbm, hdp_hbm, ms_hbm, s3_hbm, d3_hbm, out_hbm,
          hs_v, hdp_v, ms_v, s3v, d3v, rows_a, rows_b, z_v, hsp, accsp,
          sem_a, sem_b):
        core = lax.axis_index("c")
        sub = lax.axis_index("s")
        wid = sub * 2 + core
        i16 = lax.iota(jnp.int32, 16)
        zero16 = jnp.zeros((16,), _f32)

        pltpu.sync_copy(hs_hbm, hs_v)
        pltpu.sync_copy(hdp_hbm, hdp_v)
        pltpu.sync_copy(ms_hbm, ms_v)
        pltpu.sync_copy(s3_hbm.at[wid], s3v)
        pltpu.sync_copy(d3_hbm.at[wid], d3v)
        nslice = pl.ds(sub * NPT, NPT)
        pltpu.sync_copy(hp_hbm.at[nslice], hsp.at[nslice])
        for r in range(NPT):
            z_v[r, :] = zero16
        pltpu.sync_copy(z_v, accsp.at[nslice])
        plsc.subcore_barrier()

        ebase = wid * EPT2
        bufs = (rows_a, rows_b)
        sems = (sem_a, sem_b)

        maxs16 = ms_v[...]

        def do_chunk(j, rows_v, sem):
            pltpu.make_async_copy(hsp.at[s3v.at[j]], rows_v, sem).wait()
            for g in range(8):
                s16 = s3v[j, pl.ds(g * 16, 16)]
                d16 = d3v[j, pl.ds(g * 16, 16)]
                hs_g = plsc.load_gather(hs_v, [s16])
                hdp_g = plsc.load_gather(hdp_v, [d16])
                z = hs_g + hdp_g - maxs16
                lg = jnp.maximum(z, 0.2 * z)
                c_g = jnp.maximum(hdp_g, 0.2 * hdp_g)
                ex = jnp.exp(lg - c_g)
                eid = ebase + j * 128 + g * 16 + i16
                ex = jnp.where(eid < E2, ex, 0.0)
                for kk in range(16):
                    r = g * 16 + kk
                    exk = jnp.broadcast_to(ex[kk], (16,))
                    rows_v[r, :] = rows_v[r, :] * exk
            pltpu.sync_copy(rows_v, accsp.at[d3v.at[j]], add=True)

        pltpu.async_copy(hsp.at[s3v.at[0]], rows_a, sem_a)

        def pair_body(jj, carry):
            pltpu.async_copy(hsp.at[s3v.at[jj + 1]], rows_b, sem_b)
            do_chunk(jj, rows_a, sem_a)

            @pl.when(jj + 2 < NCH2)
            def _():
                pltpu.async_copy(hsp.at[s3v.at[jj + 2]], rows_a, sem_a)

            do_chunk(jj + 1, rows_b, sem_b)
            return carry

        lax.fori_loop(0, NCH2 // 2, lambda i, c: pair_body(i * 2, c), 0)
        plsc.subcore_barrier()
        pltpu.sync_copy(accsp.at[nslice],
                        out_hbm.at[pl.ds(core * NN + sub * NPT, NPT)])

    return k


def _make_edge_head_kernel():
    mesh = plsc.VectorSubcoreMesh(core_axis_name="c", subcore_axis_name="s")

    @functools.partial(
        pl.kernel, mesh=mesh,
        compiler_params=pltpu.CompilerParams(
            needs_layout_passes=False, use_tc_tiling_on_sc=False,
            skip_device_barrier=True),
        out_type=jax.ShapeDtypeStruct((EPAD * 4,), _f32),
        scratch_types=[
            pltpu.VMEM((NCH, 128), jnp.int32),
            pltpu.VMEM((NCH, 128), jnp.int32),
            pltpu.VMEM((128, DPAD), _f32),
            pltpu.VMEM((128, DPAD), _f32),
            pltpu.VMEM((128, DPAD), _f32),
            pltpu.VMEM((128, DPAD), _f32),
            pltpu.VMEM((128 * 17,), _f32),
            pltpu.VMEM((64,), _f32),
            pltpu.VMEM((16,), _f32),
            pltpu.VMEM((EPT * 4,), _f32),
            pltpu.VMEM_SHARED((NN, DPAD), _f32),
            pltpu.VMEM_SHARED((NN, DPAD), _f32),
            pltpu.SemaphoreType.DMA,
            pltpu.SemaphoreType.DMA,
        ],
    )
    def k(p_hbm, q_hbm, s3_hbm, d3_hbm, w9_hbm, b9_hbm, out_hbm,
          s3v, d3v, pr_a, qr_a, pr_b, qr_b, ut_v, w9_v, b9_v, ob_v,
          psp, qsp, sem_a, sem_b):
        core = lax.axis_index("c")
        sub = lax.axis_index("s")
        wid = sub * 2 + core
        i16 = lax.iota(jnp.int32, 16)

        pltpu.sync_copy(s3_hbm.at[wid], s3v)
        pltpu.sync_copy(d3_hbm.at[wid], d3v)
        pltpu.sync_copy(w9_hbm, w9_v)
        pltpu.sync_copy(b9_hbm, b9_v)
        nslice = pl.ds(sub * NPT, NPT)
        pltpu.sync_copy(p_hbm.at[nslice], psp.at[nslice])
        pltpu.sync_copy(q_hbm.at[nslice], qsp.at[nslice])
        plsc.subcore_barrier()

        w9cols = [w9_v[pl.ds(c * 16, 16)] for c in range(4)]
        b9all = b9_v[...]
        w9b = [[jnp.broadcast_to(w9cols[c][f], (16,)) for c in range(4)]
               for f in range(10)]
        b9b = [jnp.broadcast_to(b9all[c], (16,)) for c in range(4)]
        i16x17 = i16 * 17
        i16x4 = i16 * 4
        obase = wid * EPT * 4

        def gather_pq(j, pr_v, qr_v, sem):
            pltpu.async_copy(psp.at[d3v.at[j]], pr_v, sem)
            pltpu.async_copy(qsp.at[s3v.at[j]], qr_v, sem)

        def do_chunk(j, pr_v, qr_v, sem):
            pltpu.make_async_copy(psp.at[d3v.at[0]], pr_v, sem).wait()
            pltpu.make_async_copy(qsp.at[s3v.at[0]], qr_v, sem).wait()
            for r in range(128):
                u = jnp.maximum(pr_v[r, :] + qr_v[r, :], 0.0)
                ut_v[pl.ds(r * 17, 16)] = u
            for g in range(8):
                o = [b9b[c] for c in range(4)]
                base17 = g * 16 * 17
                for f in range(10):
                    uf = plsc.load_gather(ut_v, [base17 + i16x17 + f])
                    o = [o[c] + uf * w9b[f][c] for c in range(4)]
                o = [jnp.maximum(oc, 0.0) for oc in o]
                m = jnp.maximum(jnp.maximum(o[0], o[1]), jnp.maximum(o[2], o[3]))
                ev = [jnp.exp(oc - m) for oc in o]
                r = 1.0 / (ev[0] + ev[1] + ev[2] + ev[3])
                le4 = j * 512 + g * 64 + i16x4
                for c in range(4):
                    plsc.store_scatter(ob_v, [le4 + c], ev[c] * r)

        gather_pq(0, pr_a, qr_a, sem_a)

        def pair_body(jj, carry):
            gather_pq(jj + 1, pr_b, qr_b, sem_b)
            do_chunk(jj, pr_a, qr_a, sem_a)

            @pl.when(jj + 2 < NCH)
            def _():
                gather_pq(jj + 2, pr_a, qr_a, sem_a)

            do_chunk(jj + 1, pr_b, qr_b, sem_b)
            return carry

        lax.fori_loop(0, NCH // 2, lambda i, c: pair_body(i * 2, c), 0)
        pltpu.sync_copy(ob_v, out_hbm.at[pl.ds(obase, EPT * 4)])

    return k


_gat = _make_gat_edge_kernel()
_head = _make_edge_head_kernel()


def _padw(w, r, c):
    return jnp.zeros((r, c), _f32).at[:w.shape[0], :w.shape[1]].set(w)


def _padv(v, r):
    return jnp.zeros((r,), _f32).at[:v.shape[0]].set(v)


def kernel(x, e, W1, a1s, a1d, b1, W2, a2s, a2d, b2, W3, a3s, a3d, b3, We, be,
           W9, b9, edge_index):
    si = jnp.arange(N, dtype=jnp.int32)
    s2_full = jnp.zeros((EPAD2,), jnp.int32).at[:E].set(edge_index[0]).at[E:E2].set(si)
    d2_full = jnp.zeros((EPAD2,), jnp.int32).at[:E].set(edge_index[1]).at[E:E2].set(si)
    sa3 = s2_full.reshape(NW, NCH2, 128)
    da3 = d2_full.reshape(NW, NCH2, 128)

    s_flat = jnp.zeros((EPAD,), jnp.int32).at[:E].set(edge_index[0])
    d_flat = jnp.zeros((EPAD,), jnp.int32).at[:E].set(edge_index[1])
    s3 = s_flat.reshape(NW, NCH, 128)
    d3 = d_flat.reshape(NW, NCH, 128)

    hp, hs, hd, c = _tc_prep_x(
        x, _padw(W1, 128, DPAD), _padw(a1s[:, None], DPAD, 1),
        _padw(a1d[:, None], DPAD, 1), 5)
    acc = _gat(hp, hs.reshape(NN), hd.reshape(NN), c.reshape(DPAD), sa3, da3)

    hp, hs, hd, c = _tc_prep_acc(
        acc, _padv(b1, DPAD)[None, :], _padw(W2, DPAD, DPAD),
        _padw(a2s[:, None], DPAD, 1), _padw(a2d[:, None], DPAD, 1), 5, 10)
    acc = _gat(hp, hs.reshape(NN), hd.reshape(NN), c.reshape(DPAD), sa3, da3)

    hp, hs, hd, c = _tc_prep_acc(
        acc, _padv(b2, DPAD)[None, :], _padw(W3, DPAD, DPAD),
        _padw(a3s[:, None], DPAD, 1), _padw(a3d[:, None], DPAD, 1), 10, 10)
    acc = _gat(hp, hs.reshape(NN), hd.reshape(NN), c.reshape(DPAD), sa3, da3)

    p, q = _tc_prep_final(
        acc, _padv(b3, DPAD)[None, :], _padw(We[:10] - We[10:], DPAD, DPAD),
        _padw(We[10:], DPAD, DPAD), _padv(be, DPAD)[None, :], 10)
    w9cols = jnp.zeros((64,), _f32).at[0:10].set(W9[:, 0]).at[16:26].set(
        W9[:, 1]).at[32:42].set(W9[:, 2]).at[48:58].set(W9[:, 3])
    out = _head(p, q, s3, d3, w9cols, _padv(b9, 16))
    return out.reshape(EPAD, 4)[:E]


# speedup vs baseline: 2.0529x; 1.2150x over previous
import functools

import jax
import jax.numpy as jnp
from jax import lax
from jax.experimental import pallas as pl
from jax.experimental.pallas import tpu as pltpu
from jax.experimental.pallas import tpu_sc as plsc

N = 10000
E = 320000
DPAD = 16
NW = 32
NN = 10240
NPT = NN // 16

E2 = E + N
NCH2 = 82
EPT2 = NCH2 * 128
EPAD2 = NW * EPT2

NCH = 80
EPT = NCH * 128
EPAD = NW * EPT

_f32 = jnp.float32


def _emit_node_tables(h, as_ref, ad_ref, hp_ref, hs_ref, hdp_ref, ms_ref, d_out):
    col = lax.broadcasted_iota(jnp.int32, (N, DPAD), 1)
    hp_ref[:N, :] = h + jnp.where(col == d_out, 1.0, 0.0).astype(_f32)
    hp_ref[N:, :] = jnp.zeros((NN - N, DPAD), _f32)
    hs = jnp.dot(h, as_ref[...], preferred_element_type=_f32)
    hd = jnp.dot(h, ad_ref[...], preferred_element_type=_f32)
    hs_ref[:N, :] = hs
    hs_ref[N:, :] = jnp.zeros((NN - N, 1), _f32)
    maxs = jnp.max(hs)
    hdp_ref[:N, :] = maxs + hd
    hdp_ref[N:, :] = jnp.zeros((NN - N, 1), _f32)
    ms_ref[...] = jnp.broadcast_to(maxs, (1, DPAD))


def _prep_from_x(x_ref, w_ref, as_ref, ad_ref, hp_ref, hs_ref, hd_ref, c_ref, *, d_out):
    h = jnp.dot(x_ref[...], w_ref[...], preferred_element_type=_f32)
    _emit_node_tables(h, as_ref, ad_ref, hp_ref, hs_ref, hd_ref, c_ref, d_out)


def _prep_from_acc(acc_ref, b_ref, w_ref, as_ref, ad_ref, hp_ref, hs_ref, hd_ref,
                   c_ref, *, d_prev, d_out):
    num = acc_ref[:N, :] + acc_ref[NN:NN + N, :]
    den = num[:, d_prev:d_prev + 1] + 1e-16
    hprev = jnp.maximum(num / den + b_ref[...], 0.0)
    h = jnp.dot(hprev, w_ref[...], preferred_element_type=_f32)
    _emit_node_tables(h, as_ref, ad_ref, hp_ref, hs_ref, hd_ref, c_ref, d_out)


def _prep_final(acc_ref, b_ref, wa_ref, wb_ref, be_ref, p_ref, q_ref, *, d_prev):
    num = acc_ref[:N, :] + acc_ref[NN:NN + N, :]
    den = num[:, d_prev:d_prev + 1] + 1e-16
    h = jnp.maximum(num / den + b_ref[...], 0.0)
    p_ref[:N, :] = jnp.dot(h, wa_ref[...], preferred_element_type=_f32) + be_ref[...]
    p_ref[N:, :] = jnp.zeros((NN - N, DPAD), _f32)
    q_ref[:N, :] = jnp.dot(h, wb_ref[...], preferred_element_type=_f32)
    q_ref[N:, :] = jnp.zeros((NN - N, DPAD), _f32)


_TABLE_OUT = [
    jax.ShapeDtypeStruct((NN, DPAD), _f32),
    jax.ShapeDtypeStruct((NN, 1), _f32),
    jax.ShapeDtypeStruct((NN, 1), _f32),
    jax.ShapeDtypeStruct((1, DPAD), _f32),
]


def _tc_prep_x(x, wp, asp, adp, d_out):
    return pl.pallas_call(
        functools.partial(_prep_from_x, d_out=d_out),
        out_shape=_TABLE_OUT,
    )(x, wp, asp, adp)


def _tc_prep_acc(acc, bp, wp, asp, adp, d_prev, d_out):
    return pl.pallas_call(
        functools.partial(_prep_from_acc, d_prev=d_prev, d_out=d_out),
        out_shape=_TABLE_OUT,
    )(acc, bp, wp, asp, adp)


def _tc_prep_final(acc, bp, wap, wbp, bep, d_prev):
    return pl.pallas_call(
        functools.partial(_prep_final, d_prev=d_prev),
        out_shape=[
            jax.ShapeDtypeStruct((NN, DPAD), _f32),
            jax.ShapeDtypeStruct((NN, DPAD), _f32),
        ],
    )(acc, bp, wap, wbp, bep)


def _make_gat_edge_kernel():
    mesh = plsc.VectorSubcoreMesh(core_axis_name="c", subcore_axis_name="s")

    @functools.partial(
        pl.kernel, mesh=mesh,
        compiler_params=pltpu.CompilerParams(
            needs_layout_passes=False, use_tc_tiling_on_sc=False,
            skip_device_barrier=True),
        out_type=jax.ShapeDtypeStruct((2 * NN, DPAD), _f32),
        scratch_types=[
            pltpu.VMEM((NN,), _f32),
            pltpu.VMEM((NN,), _f32),
            pltpu.VMEM((16,), _f32),
            pltpu.VMEM((NCH2, 128), jnp.int32),
            pltpu.VMEM((NCH2, 128), jnp.int32),
            pltpu.VMEM((128, DPAD), _f32),
            pltpu.VMEM((128, DPAD), _f32),
            pltpu.VMEM((NPT, DPAD), _f32),
            pltpu.VMEM_SHARED((NN, DPAD), _f32),
            pltpu.VMEM_SHARED((NN, DPAD), _f32),
            pltpu.SemaphoreType.DMA,
            pltpu.SemaphoreType.DMA,
        ],
    )
    def k(hp_hbm, hs_hbm, hdp_hbm, ms_hbm, s3_hbm, d3_hbm, out_hbm,
          hs_v, hdp_v, ms_v, s3v, d3v, rows_a, rows_b, z_v, hsp, accsp,
          sem_a, sem_b):
        core = lax.axis_index("c")
        sub = lax.axis_index("s")
        wid = sub * 2 + core
        i16 = lax.iota(jnp.int32, 16)
        zero16 = jnp.zeros((16,), _f32)

        pltpu.sync_copy(hs_hbm, hs_v)
        pltpu.sync_copy(hdp_hbm, hdp_v)
        pltpu.sync_copy(ms_hbm, ms_v)
        pltpu.sync_copy(s3_hbm.at[wid], s3v)
        pltpu.sync_copy(d3_hbm.at[wid], d3v)
        nslice = pl.ds(sub * NPT, NPT)
        pltpu.sync_copy(hp_hbm.at[nslice], hsp.at[nslice])
        for r in range(NPT):
            z_v[r, :] = zero16
        pltpu.sync_copy(z_v, accsp.at[nslice])
        plsc.subcore_barrier()

        ebase = wid * EPT2
        bufs = (rows_a, rows_b)
        sems = (sem_a, sem_b)

        maxs16 = ms_v[...]

        def do_chunk(j, rows_v, sem):
            pltpu.make_async_copy(hsp.at[s3v.at[j]], rows_v, sem).wait()
            for g in range(8):
                s16 = s3v[j, pl.ds(g * 16, 16)]
                d16 = d3v[j, pl.ds(g * 16, 16)]
                hs_g = plsc.load_gather(hs_v, [s16])
                hdp_g = plsc.load_gather(hdp_v, [d16])
                z = hs_g + hdp_g - maxs16
                lg = jnp.maximum(z, 0.2 * z)
                c_g = jnp.maximum(hdp_g, 0.2 * hdp_g)
                ex = jnp.exp(lg - c_g)
                eid = ebase + j * 128 + g * 16 + i16
                ex = jnp.where(eid < E2, ex, 0.0)
                for kk in range(16):
                    r = g * 16 + kk
                    exk = jnp.broadcast_to(ex[kk], (16,))
                    rows_v[r, :] = rows_v[r, :] * exk
            pltpu.sync_copy(rows_v, accsp.at[d3v.at[j]], add=True)

        pltpu.async_copy(hsp.at[s3v.at[0]], rows_a, sem_a)

        def pair_body(jj, carry):
            pltpu.async_copy(hsp.at[s3v.at[jj + 1]], rows_b, sem_b)
            do_chunk(jj, rows_a, sem_a)

            @pl.when(jj + 2 < NCH2)
            def _():
                pltpu.async_copy(hsp.at[s3v.at[jj + 2]], rows_a, sem_a)

            do_chunk(jj + 1, rows_b, sem_b)
            return carry

        lax.fori_loop(0, NCH2 // 2, lambda i, c: pair_body(i * 2, c), 0)
        plsc.subcore_barrier()
        pltpu.sync_copy(accsp.at[nslice],
                        out_hbm.at[pl.ds(core * NN + sub * NPT, NPT)])

    return k


def _make_edge_head_kernel():
    mesh = plsc.VectorSubcoreMesh(core_axis_name="c", subcore_axis_name="s")

    @functools.partial(
        pl.kernel, mesh=mesh,
        compiler_params=pltpu.CompilerParams(
            needs_layout_passes=False, use_tc_tiling_on_sc=False,
            skip_device_barrier=True),
        out_type=jax.ShapeDtypeStruct((EPAD, DPAD), _f32),
        scratch_types=[
            pltpu.VMEM((NCH, 128), jnp.int32),
            pltpu.VMEM((NCH, 128), jnp.int32),
            pltpu.VMEM((128, DPAD), _f32),
            pltpu.VMEM((128, DPAD), _f32),
            pltpu.VMEM((128, DPAD), _f32),
            pltpu.VMEM((128, DPAD), _f32),
            pltpu.VMEM_SHARED((NN, DPAD), _f32),
            pltpu.VMEM_SHARED((NN, DPAD), _f32),
            pltpu.SemaphoreType.DMA,
            pltpu.SemaphoreType.DMA,
        ],
    )
    def k(p_hbm, q_hbm, s3_hbm, d3_hbm, out_hbm,
          s3v, d3v, pr_a, qr_a, pr_b, qr_b, psp, qsp, sem_a, sem_b):
        core = lax.axis_index("c")
        sub = lax.axis_index("s")
        wid = sub * 2 + core

        pltpu.sync_copy(s3_hbm.at[wid], s3v)
        pltpu.sync_copy(d3_hbm.at[wid], d3v)
        nslice = pl.ds(sub * NPT, NPT)
        pltpu.sync_copy(p_hbm.at[nslice], psp.at[nslice])
        pltpu.sync_copy(q_hbm.at[nslice], qsp.at[nslice])
        plsc.subcore_barrier()

        ebase = wid * EPT

        def gather_pq(j, pr_v, qr_v, sem):
            pltpu.async_copy(psp.at[d3v.at[j]], pr_v, sem)
            pltpu.async_copy(qsp.at[s3v.at[j]], qr_v, sem)

        def do_chunk(j, pr_v, qr_v, sem):
            pltpu.make_async_copy(psp.at[d3v.at[0]], pr_v, sem).wait()
            pltpu.make_async_copy(qsp.at[s3v.at[0]], qr_v, sem).wait()
            for r in range(128):
                pr_v[r, :] = jnp.maximum(pr_v[r, :] + qr_v[r, :], 0.0)
            pltpu.sync_copy(pr_v, out_hbm.at[pl.ds(ebase + j * 128, 128)])

        gather_pq(0, pr_a, qr_a, sem_a)

        def pair_body(jj, carry):
            gather_pq(jj + 1, pr_b, qr_b, sem_b)
            do_chunk(jj, pr_a, qr_a, sem_a)

            @pl.when(jj + 2 < NCH)
            def _():
                gather_pq(jj + 2, pr_a, qr_a, sem_a)

            do_chunk(jj + 1, pr_b, qr_b, sem_b)
            return carry

        lax.fori_loop(0, NCH // 2, lambda i, c: pair_body(i * 2, c), 0)

    return k


_HROWS = EPAD // 8
_HEAD_BLK = 4096


def _head_epilogue(u_ref, w_ref, b_ref, e_ref, out_ref):
    o = jnp.dot(u_ref[...], w_ref[...], preferred_element_type=_f32) + b_ref[...]
    o = jnp.maximum(o, 0.0)
    m = jnp.max(o, axis=1, keepdims=True)
    ev = jnp.exp(o - m)
    s = jnp.dot(ev, e_ref[...], preferred_element_type=_f32)
    out_ref[...] = ev / s


def _tc_head(u128, w9blk, b9tile, eblk):
    return pl.pallas_call(
        _head_epilogue,
        grid=(_HROWS // _HEAD_BLK,),
        in_specs=[
            pl.BlockSpec((_HEAD_BLK, 128), lambda i: (i, 0)),
            pl.BlockSpec((128, 32), lambda i: (0, 0)),
            pl.BlockSpec((1, 32), lambda i: (0, 0)),
            pl.BlockSpec((32, 32), lambda i: (0, 0)),
        ],
        out_specs=pl.BlockSpec((_HEAD_BLK, 32), lambda i: (i, 0)),
        out_shape=jax.ShapeDtypeStruct((_HROWS, 32), _f32),
    )(u128, w9blk, b9tile, eblk)


_gat = _make_gat_edge_kernel()
_head = _make_edge_head_kernel()


def _padw(w, r, c):
    return jnp.zeros((r, c), _f32).at[:w.shape[0], :w.shape[1]].set(w)


def _padv(v, r):
    return jnp.zeros((r,), _f32).at[:v.shape[0]].set(v)


def kernel(x, e, W1, a1s, a1d, b1, W2, a2s, a2d, b2, W3, a3s, a3d, b3, We, be,
           W9, b9, edge_index):
    si = jnp.arange(N, dtype=jnp.int32)
    s2_full = jnp.zeros((EPAD2,), jnp.int32).at[:E].set(edge_index[0]).at[E:E2].set(si)
    d2_full = jnp.zeros((EPAD2,), jnp.int32).at[:E].set(edge_index[1]).at[E:E2].set(si)
    sa3 = s2_full.reshape(NW, NCH2, 128)
    da3 = d2_full.reshape(NW, NCH2, 128)

    s_flat = jnp.zeros((EPAD,), jnp.int32).at[:E].set(edge_index[0])
    d_flat = jnp.zeros((EPAD,), jnp.int32).at[:E].set(edge_index[1])
    s3 = s_flat.reshape(NW, NCH, 128)
    d3 = d_flat.reshape(NW, NCH, 128)

    hp, hs, hd, c = _tc_prep_x(
        x, _padw(W1, 128, DPAD), _padw(a1s[:, None], DPAD, 1),
        _padw(a1d[:, None], DPAD, 1), 5)
    acc = _gat(hp, hs.reshape(NN), hd.reshape(NN), c.reshape(DPAD), sa3, da3)

    hp, hs, hd, c = _tc_prep_acc(
        acc, _padv(b1, DPAD)[None, :], _padw(W2, DPAD, DPAD),
        _padw(a2s[:, None], DPAD, 1), _padw(a2d[:, None], DPAD, 1), 5, 10)
    acc = _gat(hp, hs.reshape(NN), hd.reshape(NN), c.reshape(DPAD), sa3, da3)

    hp, hs, hd, c = _tc_prep_acc(
        acc, _padv(b2, DPAD)[None, :], _padw(W3, DPAD, DPAD),
        _padw(a3s[:, None], DPAD, 1), _padw(a3d[:, None], DPAD, 1), 10, 10)
    acc = _gat(hp, hs.reshape(NN), hd.reshape(NN), c.reshape(DPAD), sa3, da3)

    p, q = _tc_prep_final(
        acc, _padv(b3, DPAD)[None, :], _padw(We[:10] - We[10:], DPAD, DPAD),
        _padw(We[10:], DPAD, DPAD), _padv(be, DPAD)[None, :], 10)
    u = _head(p, q, s3, d3)
    w9blk = jnp.kron(jnp.eye(8, dtype=_f32), _padw(W9, DPAD, 4))
    b9tile = jnp.tile(b9, 8)[None, :]
    eblk = jnp.kron(jnp.eye(8, dtype=_f32), jnp.ones((4, 4), _f32))
    out32 = _tc_head(u.reshape(_HROWS, 128), w9blk, b9tile, eblk)
    return out32.reshape(EPAD, 4)[:E]
